# baseline (device time: 305604 ns/iter reference)
import jax
import jax.numpy as jnp
from jax import lax
from jax.experimental import pallas as pl
from jax.experimental.pallas import tpu as pltpu

N_DEV = 32
M_PER = 64
HALF = M_PER // 2
D = 1024

R_HOPS = N_DEV // 2
L_HOPS = N_DEV // 2 - 1

CYCLE = [0, 3, 4, 7, 15, 12, 11, 8, 16, 19, 20, 23, 31, 28, 27, 24,
         25, 26, 29, 30, 22, 21, 18, 17, 9, 10, 13, 14, 6, 5, 2, 1]


def kernel(x, Win0, Wout0, Win1, Wout1, Win2, Wout2):
    xb = x.astype(jnp.bfloat16)
    wins = [w.astype(jnp.bfloat16) for w in (Win0, Win1, Win2)]
    wouts = [w.astype(jnp.bfloat16) for w in (Wout0, Wout1, Wout2)]
    cyc_tbl = jnp.asarray(CYCLE, dtype=jnp.int32).reshape(1, N_DEV)
    inv = [0] * N_DEV
    for k, v in enumerate(CYCLE):
        inv[v] = k
    inv_tbl = jnp.asarray(inv, dtype=jnp.int32).reshape(1, N_DEV)

    def body(cyc_ref, inv_ref, x_ref, win0_ref, win1_ref, win2_ref,
             wout0_ref, wout1_ref, wout2_ref, out_ref,
             xfull_ref, p_ref, sbuf_r_ref, sbuf_l_ref,
             rsr_ref, rsl_ref,
             ag_sr_sems, ag_sl_sems, rs_sr_sems, rs_sl_sems,
             ag_r_sems, ag_l_sems, rs_r_sems, rs_l_sems):
        me = lax.axis_index("i")

        def cyc(k):
            return cyc_ref[0, lax.rem(k + 2 * N_DEV, N_DEV)]

        c_me = inv_ref[0, me]
        left = cyc(c_me - 1)
        right = cyc(c_me + 1)

        barrier_sem = pltpu.get_barrier_semaphore()
        for nbr in (left, right):
            pl.semaphore_signal(
                barrier_sem, inc=1,
                device_id=(nbr,), device_id_type=pl.DeviceIdType.MESH,
            )
        pl.semaphore_wait(barrier_sem, 2)

        win_refs = (win0_ref, win1_ref, win2_ref)
        wout_refs = (wout0_ref, wout1_ref, wout2_ref)

        def ag_rdma(direction, half, h):
            if direction == 0:
                bj = cyc(c_me - h)
                tgt, ssem, rsem = right, ag_sr_sems, ag_r_sems
                n_steps = R_HOPS
            else:
                bj = cyc(c_me + h)
                tgt, ssem, rsem = left, ag_sl_sems, ag_l_sems
                n_steps = L_HOPS
            rows = pl.ds(bj * M_PER + half * HALF, HALF)
            return pltpu.make_async_remote_copy(
                src_ref=xfull_ref.at[rows],
                dst_ref=xfull_ref.at[rows],
                send_sem=ssem.at[half],
                recv_sem=rsem.at[half * n_steps + h],
                device_id=(tgt,),
                device_id_type=pl.DeviceIdType.MESH,
            )

        def rs_rdma(direction, half, s):
            if direction == 0:
                tgt, ssem, rsem = right, rs_sr_sems, rs_r_sems
                sbuf, slots, n_steps = sbuf_r_ref, rsr_ref, L_HOPS
            else:
                tgt, ssem, rsem = left, rs_sl_sems, rs_l_sems
                sbuf, slots, n_steps = sbuf_l_ref, rsl_ref, R_HOPS
            return pltpu.make_async_remote_copy(
                src_ref=sbuf.at[pl.ds(half * HALF, HALF)],
                dst_ref=slots.at[pl.ds(s * M_PER + half * HALF, HALF)],
                send_sem=ssem.at[half],
                recv_sem=rsem.at[half * n_steps + s],
                device_id=(tgt,),
                device_id_type=pl.DeviceIdType.MESH,
            )

        def compute_block(b, l):
            rows = pl.ds(b * M_PER, M_PER)
            xs = xfull_ref[rows, :]
            hh = jnp.dot(xs, win_refs[l][:, :],
                         preferred_element_type=jnp.float32)
            hh = jnp.maximum(hh, 0.0).astype(jnp.bfloat16)
            p_ref[rows, :] = jnp.dot(hh, wout_refs[l][:, :],
                                     preferred_element_type=jnp.float32)

        xfull_ref[pl.ds(me * M_PER, M_PER), :] = x_ref[:, :]

        for l in range(3):
            def ag_step(h, carry):
                for half in range(2):
                    @pl.when(h > 0)
                    def _():
                        ag_rdma(0, half, h - 1).wait()
                    ag_rdma(0, half, h).start()
                    @pl.when(h > 0)
                    def _():
                        ag_rdma(1, half, h - 1).wait()

                    @pl.when(h < L_HOPS)
                    def _():
                        ag_rdma(1, half, h).start()

                @pl.when(h >= R_HOPS // 2)
                def _():
                    compute_block(cyc(c_me - h), l)
                    compute_block(cyc(c_me + h), l)

                return carry

            lax.fori_loop(0, R_HOPS, ag_step, 0, unroll=False)
            for half in range(2):
                ag_rdma(0, half, R_HOPS - 1).wait()
            compute_block(cyc(c_me - R_HOPS), l)

            def rs_val(direction, half, s):
                if direction == 0:
                    bi = cyc(c_me + L_HOPS - s)
                    slots = rsr_ref
                else:
                    bi = cyc(c_me - R_HOPS + s)
                    slots = rsl_ref
                rows = pl.ds(bi * M_PER + half * HALF, HALF)
                own = p_ref[rows, :]
                sprev = jnp.maximum(s - 1, 0)
                prev = slots[pl.ds(sprev * M_PER + half * HALF, HALF), :]
                return jnp.where(
                    s == 0, own, own + prev.astype(jnp.float32)
                ).astype(jnp.bfloat16)

            def rs_step(s, carry):
                for half in range(2):
                    @pl.when(s > 0)
                    def _():
                        rs_rdma(1, half, s - 1).wait()
                    sbuf_l_ref[pl.ds(half * HALF, HALF), :] = rs_val(1, half, s)
                    rs_rdma(1, half, s).start()
                    @pl.when(s > 0)
                    def _():
                        rs_rdma(0, half, s - 1).wait()

                    @pl.when(s < L_HOPS)
                    def _():
                        sbuf_r_ref[pl.ds(half * HALF, HALF), :] = rs_val(
                            0, half, s)
                        rs_rdma(0, half, s).start()

                @pl.when(s < R_HOPS // 2 - 1)
                def _():
                    compute_block(cyc(c_me + (R_HOPS // 2 - 1) - s), l)
                    compute_block(cyc(c_me - (R_HOPS // 2 - 1) + s), l)

                @pl.when(s == R_HOPS // 2 - 1)
                def _():
                    compute_block(me, l)

                return carry

            lax.fori_loop(0, R_HOPS, rs_step, 0, unroll=False)
            for half in range(2):
                rs_rdma(1, half, R_HOPS - 1).wait()

            xnew = (p_ref[pl.ds(me * M_PER, M_PER), :]
                    + rsr_ref[pl.ds((L_HOPS - 1) * M_PER, M_PER),
                              :].astype(jnp.float32)
                    + rsl_ref[pl.ds((R_HOPS - 1) * M_PER, M_PER),
                              :].astype(jnp.float32))
            if l < 2:
                xfull_ref[pl.ds(me * M_PER, M_PER), :] = xnew.astype(
                    jnp.bfloat16)
            else:
                out_ref[:, :] = xnew

    return pl.pallas_call(
        body,
        out_shape=jax.ShapeDtypeStruct((M_PER, D), jnp.float32),
        in_specs=[pl.BlockSpec(memory_space=pltpu.SMEM)] * 2
        + [pl.BlockSpec(memory_space=pltpu.VMEM)] * 7,
        out_specs=pl.BlockSpec(memory_space=pltpu.VMEM),
        scratch_shapes=[
            pltpu.VMEM((N_DEV * M_PER, D), jnp.bfloat16),
            pltpu.VMEM((N_DEV * M_PER, D), jnp.float32),
            pltpu.VMEM((M_PER, D), jnp.bfloat16),
            pltpu.VMEM((M_PER, D), jnp.bfloat16),
            pltpu.VMEM((L_HOPS * M_PER, D), jnp.bfloat16),
            pltpu.VMEM((R_HOPS * M_PER, D), jnp.bfloat16),
            pltpu.SemaphoreType.DMA((2,)),
            pltpu.SemaphoreType.DMA((2,)),
            pltpu.SemaphoreType.DMA((2,)),
            pltpu.SemaphoreType.DMA((2,)),
            pltpu.SemaphoreType.DMA((2 * R_HOPS,)),
            pltpu.SemaphoreType.DMA((2 * L_HOPS,)),
            pltpu.SemaphoreType.DMA((2 * L_HOPS,)),
            pltpu.SemaphoreType.DMA((2 * R_HOPS,)),
        ],
        compiler_params=pltpu.CompilerParams(collective_id=0),
    )(cyc_tbl, inv_tbl, xb, wins[0], wins[1], wins[2],
      wouts[0], wouts[1], wouts[2])


# device time: 304183 ns/iter; 1.0047x vs baseline; 1.0047x over previous
import jax
import jax.numpy as jnp
from jax import lax
from jax.experimental import pallas as pl
from jax.experimental.pallas import tpu as pltpu

N_DEV = 32
M_PER = 64
HALF = M_PER // 2
D = 1024

R_HOPS = N_DEV // 2
L_HOPS = N_DEV // 2 - 1

CYCLE = [0, 3, 4, 7, 15, 12, 11, 8, 16, 19, 20, 23, 31, 28, 27, 24,
         25, 26, 29, 30, 22, 21, 18, 17, 9, 10, 13, 14, 6, 5, 2, 1]


def kernel(x, Win0, Wout0, Win1, Wout1, Win2, Wout2):
    xb = x.astype(jnp.bfloat16)
    wins = [w.astype(jnp.bfloat16) for w in (Win0, Win1, Win2)]
    wouts = [w.astype(jnp.bfloat16) for w in (Wout0, Wout1, Wout2)]
    cyc_tbl = jnp.asarray(CYCLE, dtype=jnp.int32).reshape(1, N_DEV)
    inv = [0] * N_DEV
    for k, v in enumerate(CYCLE):
        inv[v] = k
    inv_tbl = jnp.asarray(inv, dtype=jnp.int32).reshape(1, N_DEV)

    def body(cyc_ref, inv_ref, x_ref, win0_ref, win1_ref, win2_ref,
             wout0_ref, wout1_ref, wout2_ref, out_ref,
             xfull_ref, p_ref, sbuf_r_ref, sbuf_l_ref,
             rsr_ref, rsl_ref,
             ag_sr_sems, ag_sl_sems, rs_sr_sems, rs_sl_sems,
             ag_r_sems, ag_l_sems, rs_r_sems, rs_l_sems):
        me = lax.axis_index("i")

        def cyc(k):
            return cyc_ref[0, lax.rem(k + 2 * N_DEV, N_DEV)]

        c_me = inv_ref[0, me]
        left = cyc(c_me - 1)
        right = cyc(c_me + 1)

        barrier_sem = pltpu.get_barrier_semaphore()
        for nbr in (left, right):
            pl.semaphore_signal(
                barrier_sem, inc=1,
                device_id=(nbr,), device_id_type=pl.DeviceIdType.MESH,
            )
        pl.semaphore_wait(barrier_sem, 2)

        win_refs = (win0_ref, win1_ref, win2_ref)
        wout_refs = (wout0_ref, wout1_ref, wout2_ref)

        def ag_rdma(direction, half, h):
            if direction == 0:
                bj = cyc(c_me - h)
                tgt, ssem, rsem = right, ag_sr_sems, ag_r_sems
                n_steps = R_HOPS
            else:
                bj = cyc(c_me + h)
                tgt, ssem, rsem = left, ag_sl_sems, ag_l_sems
                n_steps = L_HOPS
            rows = pl.ds(bj * M_PER + half * HALF, HALF)
            return pltpu.make_async_remote_copy(
                src_ref=xfull_ref.at[rows],
                dst_ref=xfull_ref.at[rows],
                send_sem=ssem.at[half],
                recv_sem=rsem.at[half * n_steps + h],
                device_id=(tgt,),
                device_id_type=pl.DeviceIdType.MESH,
            )

        def rs_rdma(direction, half, s):
            if direction == 0:
                tgt, ssem, rsem = right, rs_sr_sems, rs_r_sems
                sbuf, slots, n_steps = sbuf_r_ref, rsr_ref, L_HOPS
            else:
                tgt, ssem, rsem = left, rs_sl_sems, rs_l_sems
                sbuf, slots, n_steps = sbuf_l_ref, rsl_ref, R_HOPS
            return pltpu.make_async_remote_copy(
                src_ref=sbuf.at[pl.ds(half * HALF, HALF)],
                dst_ref=slots.at[pl.ds(s * M_PER + half * HALF, HALF)],
                send_sem=ssem.at[half],
                recv_sem=rsem.at[half * n_steps + s],
                device_id=(tgt,),
                device_id_type=pl.DeviceIdType.MESH,
            )

        def compute_block(b, l):
            rows = pl.ds(b * M_PER, M_PER)
            xs = xfull_ref[rows, :]
            hh = jnp.dot(xs, win_refs[l][:, :],
                         preferred_element_type=jnp.float32)
            hh = jnp.maximum(hh, 0.0).astype(jnp.bfloat16)
            p_ref[rows, :] = jnp.dot(hh, wout_refs[l][:, :],
                                     preferred_element_type=jnp.float32)

        xfull_ref[pl.ds(me * M_PER, M_PER), :] = x_ref[:, :]

        for l in range(3):
            def ag_step(h, carry):
                for half in range(2):
                    @pl.when(h > 0)
                    def _():
                        ag_rdma(0, half, h - 1).wait()
                    ag_rdma(0, half, h).start()
                    @pl.when(h > 0)
                    def _():
                        ag_rdma(1, half, h - 1).wait()

                    @pl.when(h < L_HOPS)
                    def _():
                        ag_rdma(1, half, h).start()

                @pl.when(h == 0)
                def _():
                    compute_block(me, l)

                @pl.when(h > 0)
                def _():
                    compute_block(cyc(c_me - h), l)
                    compute_block(cyc(c_me + h), l)

                return carry

            lax.fori_loop(0, R_HOPS, ag_step, 0, unroll=False)
            for half in range(2):
                ag_rdma(0, half, R_HOPS - 1).wait()
            compute_block(cyc(c_me - R_HOPS), l)

            def rs_val(direction, half, s):
                if direction == 0:
                    bi = cyc(c_me + L_HOPS - s)
                    slots = rsr_ref
                else:
                    bi = cyc(c_me - R_HOPS + s)
                    slots = rsl_ref
                rows = pl.ds(bi * M_PER + half * HALF, HALF)
                own = p_ref[rows, :]
                sprev = jnp.maximum(s - 1, 0)
                prev = slots[pl.ds(sprev * M_PER + half * HALF, HALF), :]
                return jnp.where(
                    s == 0, own, own + prev.astype(jnp.float32)
                ).astype(jnp.bfloat16)

            def rs_step(s, carry):
                for half in range(2):
                    @pl.when(s > 0)
                    def _():
                        rs_rdma(1, half, s - 1).wait()
                    sbuf_l_ref[pl.ds(half * HALF, HALF), :] = rs_val(1, half, s)
                    rs_rdma(1, half, s).start()
                    @pl.when(s > 0)
                    def _():
                        rs_rdma(0, half, s - 1).wait()

                    @pl.when(s < L_HOPS)
                    def _():
                        sbuf_r_ref[pl.ds(half * HALF, HALF), :] = rs_val(
                            0, half, s)
                        rs_rdma(0, half, s).start()

                return carry

            lax.fori_loop(0, R_HOPS, rs_step, 0, unroll=False)
            for half in range(2):
                rs_rdma(1, half, R_HOPS - 1).wait()

            xnew = (p_ref[pl.ds(me * M_PER, M_PER), :]
                    + rsr_ref[pl.ds((L_HOPS - 1) * M_PER, M_PER),
                              :].astype(jnp.float32)
                    + rsl_ref[pl.ds((R_HOPS - 1) * M_PER, M_PER),
                              :].astype(jnp.float32))
            if l < 2:
                xfull_ref[pl.ds(me * M_PER, M_PER), :] = xnew.astype(
                    jnp.bfloat16)
            else:
                out_ref[:, :] = xnew

    return pl.pallas_call(
        body,
        out_shape=jax.ShapeDtypeStruct((M_PER, D), jnp.float32),
        in_specs=[pl.BlockSpec(memory_space=pltpu.SMEM)] * 2
        + [pl.BlockSpec(memory_space=pltpu.VMEM)] * 7,
        out_specs=pl.BlockSpec(memory_space=pltpu.VMEM),
        scratch_shapes=[
            pltpu.VMEM((N_DEV * M_PER, D), jnp.bfloat16),
            pltpu.VMEM((N_DEV * M_PER, D), jnp.float32),
            pltpu.VMEM((M_PER, D), jnp.bfloat16),
            pltpu.VMEM((M_PER, D), jnp.bfloat16),
            pltpu.VMEM((L_HOPS * M_PER, D), jnp.bfloat16),
            pltpu.VMEM((R_HOPS * M_PER, D), jnp.bfloat16),
            pltpu.SemaphoreType.DMA((2,)),
            pltpu.SemaphoreType.DMA((2,)),
            pltpu.SemaphoreType.DMA((2,)),
            pltpu.SemaphoreType.DMA((2,)),
            pltpu.SemaphoreType.DMA((2 * R_HOPS,)),
            pltpu.SemaphoreType.DMA((2 * L_HOPS,)),
            pltpu.SemaphoreType.DMA((2 * L_HOPS,)),
            pltpu.SemaphoreType.DMA((2 * R_HOPS,)),
        ],
        compiler_params=pltpu.CompilerParams(collective_id=0),
    )(cyc_tbl, inv_tbl, xb, wins[0], wins[1], wins[2],
      wouts[0], wouts[1], wouts[2])
